# Initial kernel scaffold; baseline (speedup 1.0000x reference)
import jax
import jax.numpy as jnp
from jax.experimental import pallas as pl


def kernel(x, task_full, gate_w, gate_b, expert_w):
    raise NotImplementedError("write your pallas kernel here")


# speedup vs baseline: 4.2461x; 4.2461x over previous
import jax
import jax.numpy as jnp
from jax.experimental import pallas as pl
from jax.experimental.pallas import tpu as pltpu

E = 16
K = 8


def _gating_kernel(task_ref, gw_ref, gb_ref,
                   probs_ref, mask_ref, sel_idx_ref, sel_gate_ref):
    t = task_ref[...]
    h = t * jax.nn.sigmoid(t)
    logits = jnp.dot(h, gw_ref[...], preferred_element_type=jnp.float32)
    logits = logits + gb_ref[...]
    m = jnp.max(logits, axis=1, keepdims=True)
    ex = jnp.exp(logits - m)
    p = ex / jnp.sum(ex, axis=1, keepdims=True)
    probs_ref[...] = p

    col = jax.lax.broadcasted_iota(jnp.int32, (E, E), 1)
    rank = jnp.zeros((E, E), jnp.int32)
    for j in range(E):
        pj = p[:, j:j + 1]
        gt = (pj > p).astype(jnp.int32)
        eq = jnp.logical_and(pj == p, col > j).astype(jnp.int32)
        rank = rank + gt + eq
    mask = (rank < K).astype(jnp.float32)
    mask_ref[...] = mask

    m0 = mask[0:1, :]
    c0 = p[0:1, :] * m0
    row = jax.lax.broadcasted_iota(jnp.int32, (E, E), 0)
    tri = (row <= col).astype(jnp.float32)
    pos = jnp.dot(m0, tri, preferred_element_type=jnp.float32) - 1.0
    kk = jax.lax.broadcasted_iota(jnp.int32, (K, E), 0).astype(jnp.float32)
    pos_b = jnp.broadcast_to(pos, (K, E))
    onehot = jnp.where(
        jnp.logical_and(pos_b == kk, jnp.broadcast_to(m0, (K, E)) > 0.5),
        1.0, 0.0)
    cols_f = jax.lax.broadcasted_iota(jnp.int32, (K, E), 1).astype(jnp.float32)
    sel_idx_ref[...] = jnp.sum(onehot * cols_f, axis=1,
                               keepdims=True).astype(jnp.int32)
    sel_gate_ref[...] = jnp.sum(onehot * jnp.broadcast_to(c0, (K, E)),
                                axis=1, keepdims=True)


def _combine_kernel(sel_ref, gate_ref, w_ref, out_ref):
    k = pl.program_id(0)

    @pl.when(k == 0)
    def _():
        out_ref[...] = jnp.zeros_like(out_ref)

    out_ref[...] += gate_ref[k] * w_ref[0]


def _matmul_kernel(x_ref, w_ref, o_ref):
    o_ref[...] = 1.0 + jnp.dot(x_ref[...], w_ref[...],
                               preferred_element_type=jnp.float32)


def kernel(x, task_full, gate_w, gate_b, expert_w):
    B, L, D = x.shape

    probs, mask, sel_idx, sel_gate = pl.pallas_call(
        _gating_kernel,
        out_shape=(
            jax.ShapeDtypeStruct((E, E), jnp.float32),
            jax.ShapeDtypeStruct((E, E), jnp.float32),
            jax.ShapeDtypeStruct((K, 1), jnp.int32),
            jax.ShapeDtypeStruct((K, 1), jnp.float32),
        ),
    )(task_full, gate_w, gate_b.reshape(1, E))

    sel_idx = sel_idx.reshape(K)
    sel_gate = sel_gate.reshape(K)

    w_comb = pl.pallas_call(
        _combine_kernel,
        grid_spec=pltpu.PrefetchScalarGridSpec(
            num_scalar_prefetch=1,
            grid=(K,),
            in_specs=[
                pl.BlockSpec(memory_space=pltpu.SMEM),
                pl.BlockSpec((1, D, D), lambda k, sel: (sel[k], 0, 0)),
            ],
            out_specs=pl.BlockSpec((D, D), lambda k, sel: (0, 0)),
        ),
        out_shape=jax.ShapeDtypeStruct((D, D), jnp.float32),
    )(sel_idx, sel_gate, expert_w)

    BM = 512
    y0 = pl.pallas_call(
        _matmul_kernel,
        grid=(L // BM,),
        in_specs=[
            pl.BlockSpec((BM, D), lambda m: (m, 0)),
            pl.BlockSpec((D, D), lambda m: (0, 0)),
        ],
        out_specs=pl.BlockSpec((BM, D), lambda m: (m, 0)),
        out_shape=jax.ShapeDtypeStruct((L, D), jnp.float32),
        compiler_params=pltpu.CompilerParams(
            dimension_semantics=("arbitrary",)),
    )(x[0], w_comb)

    out = jnp.concatenate(
        [y0[None], jnp.ones((B - 1, L, D), jnp.float32)], axis=0)
    return out, probs[0], mask


# speedup vs baseline: 4.5506x; 1.0717x over previous
import jax
import jax.numpy as jnp
from jax.experimental import pallas as pl
from jax.experimental.pallas import tpu as pltpu

E = 16
K = 8


def _gating_kernel(task_ref, gw_ref, gb_ref,
                   probs_ref, mask_ref, sel_idx_ref, sel_gate_ref):
    t = task_ref[...]
    h = t * jax.nn.sigmoid(t)
    logits = jnp.dot(h, gw_ref[...], preferred_element_type=jnp.float32)
    logits = logits + gb_ref[...]
    m = jnp.max(logits, axis=1, keepdims=True)
    ex = jnp.exp(logits - m)
    p = ex / jnp.sum(ex, axis=1, keepdims=True)
    probs_ref[...] = p

    col = jax.lax.broadcasted_iota(jnp.int32, (E, E), 1)
    rank = jnp.zeros((E, E), jnp.int32)
    for j in range(E):
        pj = p[:, j:j + 1]
        gt = (pj > p).astype(jnp.int32)
        eq = jnp.logical_and(pj == p, col > j).astype(jnp.int32)
        rank = rank + gt + eq
    mask = (rank < K).astype(jnp.float32)
    mask_ref[...] = mask

    m0 = mask[0:1, :]
    c0 = p[0:1, :] * m0
    row = jax.lax.broadcasted_iota(jnp.int32, (E, E), 0)
    tri = (row <= col).astype(jnp.float32)
    pos = jnp.dot(m0, tri, preferred_element_type=jnp.float32) - 1.0
    kk = jax.lax.broadcasted_iota(jnp.int32, (K, E), 0).astype(jnp.float32)
    pos_b = jnp.broadcast_to(pos, (K, E))
    onehot = jnp.where(
        jnp.logical_and(pos_b == kk, jnp.broadcast_to(m0, (K, E)) > 0.5),
        1.0, 0.0)
    cols_f = jax.lax.broadcasted_iota(jnp.int32, (K, E), 1).astype(jnp.float32)
    sel_idx_ref[...] = jnp.sum(onehot * cols_f, axis=1,
                               keepdims=True).astype(jnp.int32)
    sel_gate_ref[...] = jnp.sum(onehot * jnp.broadcast_to(c0, (K, E)),
                                axis=1, keepdims=True)


def _combine_kernel(sel_ref, gate_ref, w_ref, out_ref, acc_ref):
    k = pl.program_id(0)

    @pl.when(k == 0)
    def _():
        acc_ref[...] = jnp.zeros_like(acc_ref)

    acc_ref[...] += gate_ref[k] * w_ref[0]

    @pl.when(k == K - 1)
    def _():
        out_ref[...] = acc_ref[...].astype(jnp.bfloat16)


def _matmul_kernel(x_ref, w_ref, o_ref):
    o_ref[...] = 1.0 + jnp.dot(x_ref[...], w_ref[...],
                               preferred_element_type=jnp.float32)


def kernel(x, task_full, gate_w, gate_b, expert_w):
    B, L, D = x.shape

    probs, mask, sel_idx, sel_gate = pl.pallas_call(
        _gating_kernel,
        out_shape=(
            jax.ShapeDtypeStruct((E, E), jnp.float32),
            jax.ShapeDtypeStruct((E, E), jnp.float32),
            jax.ShapeDtypeStruct((K, 1), jnp.int32),
            jax.ShapeDtypeStruct((K, 1), jnp.float32),
        ),
    )(task_full, gate_w, gate_b.reshape(1, E))

    sel_idx = sel_idx.reshape(K)
    sel_gate = sel_gate.reshape(K)

    w_comb = pl.pallas_call(
        _combine_kernel,
        grid_spec=pltpu.PrefetchScalarGridSpec(
            num_scalar_prefetch=1,
            grid=(K,),
            in_specs=[
                pl.BlockSpec(memory_space=pltpu.SMEM),
                pl.BlockSpec((1, D, D), lambda k, sel: (sel[k], 0, 0)),
            ],
            out_specs=pl.BlockSpec((D, D), lambda k, sel: (0, 0)),
            scratch_shapes=[pltpu.VMEM((D, D), jnp.float32)],
        ),
        out_shape=jax.ShapeDtypeStruct((D, D), jnp.bfloat16),
    )(sel_idx, sel_gate, expert_w)

    BM = 512
    y0 = pl.pallas_call(
        _matmul_kernel,
        grid=(L // BM,),
        in_specs=[
            pl.BlockSpec((BM, D), lambda m: (m, 0)),
            pl.BlockSpec((D, D), lambda m: (0, 0)),
        ],
        out_specs=pl.BlockSpec((BM, D), lambda m: (m, 0)),
        out_shape=jax.ShapeDtypeStruct((L, D), jnp.float32),
        compiler_params=pltpu.CompilerParams(
            dimension_semantics=("arbitrary",)),
    )(x[0].astype(jnp.bfloat16), w_comb)

    out = jnp.concatenate(
        [y0[None], jnp.ones((B - 1, L, D), jnp.float32)], axis=0)
    return out, probs[0], mask
